# baseline (speedup 1.0000x reference)
import jax
import jax.numpy as jnp
from jax import lax
from jax.experimental import pallas as pl
from jax.experimental.pallas import tpu as pltpu
from jax.experimental.pallas import tpu_sc as plsc

N = 10000
E = 320000
D = 128
DE = 16
H = 128

NC = 2
NS = 16
NW = NC * NS
C = 80
NCHUNKS = E // C
CPW = NCHUNKS // NW
RPS = N // NS
NBUF = 3
EBUF = 5

_mesh = plsc.VectorSubcoreMesh(core_axis_name="c", subcore_axis_name="s")
_sc_params = pltpu.CompilerParams(use_tc_tiling_on_sc=False)


MB = 2


EPW = E // NW


def _zero_rows(buf, nrows):
  zv = jnp.zeros((16,), jnp.float32)

  def zrow(r, carry):
    for k in range(8):
      buf[r, pl.ds(k * 16, 16)] = zv
    return carry

  lax.fori_loop(0, nrows, zrow, 0)


def _zero_stripe(sh, base, zsrc, width):
  for k in range(RPS // C):
    pltpu.sync_copy(zsrc, sh.at[pl.ds(base + k * C, C)])
  rem = RPS % C
  if rem:
    pltpu.sync_copy(zsrc.at[pl.ds(0, rem)],
                    sh.at[pl.ds(base + (RPS // C) * C, rem)])


def _sc_pass1(x, ei, ea, zd, ones):
  def body(x_hbm, ei_hbm, ea_hbm, zd_hbm, ones_hbm,
           s_out, t_out, d_out,
           src_v, dst_v, rows_v, ea_v, ones_v,
           rsem0, rsem1, ssem0, ssem1, esem0, esem1, tsem0, tsem1,
           osem0, osem1, dsem,
           s_sh, t_sh, d_sh):
    c = lax.axis_index("c")
    s = lax.axis_index("s")
    wid = c * NS + s
    rsems = (rsem0, rsem1)
    ssems = (ssem0, ssem1)
    esems = (esem0, esem1)
    tsems = (tsem0, tsem1)
    osems = (osem0, osem1)

    _zero_rows(rows_v.at[0], C)

    def zear(r, carry):
      ea_v[0, r, :] = jnp.zeros((16,), jnp.float32)
      return carry

    lax.fori_loop(0, C, zear, 0)

    @pl.when(c == 0)
    def _():
      pltpu.sync_copy(x_hbm.at[pl.ds(s * RPS, RPS)], s_sh.at[pl.ds(s * RPS, RPS)])

    @pl.when(c != 0)
    def _():
      _zero_stripe(s_sh, s * RPS, rows_v.at[0], D)

    _zero_stripe(t_sh, s * RPS, ea_v.at[0], DE)
    pltpu.sync_copy(zd_hbm.at[pl.ds(s * RPS, RPS)], d_sh.at[pl.ds(s * RPS, RPS)])
    pltpu.sync_copy(ei_hbm.at[0, pl.ds(wid * EPW, EPW)], src_v)
    pltpu.sync_copy(ones_hbm, ones_v)
    plsc.subcore_barrier()

    def batch(i0, nb):
      dds = [pltpu.async_copy(ei_hbm.at[1, pl.ds((wid * CPW + i0 + b) * C, C)],
                              dst_v.at[b], dsem) for b in range(nb)]
      rds = [pltpu.async_copy(x_hbm.at[src_v.at[pl.ds((i0 + b) * C, C)]],
                              rows_v.at[b], rsems[b]) for b in range(nb)]
      eds = [pltpu.async_copy(ea_hbm.at[pl.ds((wid * CPW + i0 + b) * C, C)],
                              ea_v.at[b], esems[b]) for b in range(nb)]
      for dd in dds:
        dd.wait()
      sds = []
      for b in range(nb):
        rds[b].wait()
        sds.append(pltpu.async_copy(rows_v.at[b], s_sh.at[dst_v.at[b]],
                                    ssems[b], add=True))
        eds[b].wait()
        sds.append(pltpu.async_copy(ea_v.at[b], t_sh.at[dst_v.at[b]],
                                    tsems[b], add=True))
        sds.append(pltpu.async_copy(ones_v, d_sh.at[dst_v.at[b]],
                                    osems[b], add=True))
      for sd in sds:
        sd.wait()

    def outer(i, carry):
      batch(i * MB, MB)
      return carry

    lax.fori_loop(0, CPW // MB, outer, 0)
    if CPW % MB:
      batch(CPW - CPW % MB, CPW % MB)

    plsc.subcore_barrier()
    pltpu.sync_copy(s_sh.at[pl.ds(s * RPS, RPS)], s_out.at[c, pl.ds(s * RPS, RPS)])
    pltpu.sync_copy(t_sh.at[pl.ds(s * RPS, RPS)], t_out.at[c, pl.ds(s * RPS, RPS)])
    pltpu.sync_copy(d_sh.at[pl.ds(s * RPS, RPS)], d_out.at[c, pl.ds(s * RPS, RPS)])

  fn = pl.kernel(
      body,
      out_type=[
          jax.ShapeDtypeStruct((NC, N, D), jnp.float32),
          jax.ShapeDtypeStruct((NC, N, DE), jnp.float32),
          jax.ShapeDtypeStruct((NC, N, 8), jnp.float32),
      ],
      mesh=_mesh,
      compiler_params=_sc_params,
      scratch_types=[
          pltpu.VMEM((EPW,), jnp.int32),
          pltpu.VMEM((MB, C), jnp.int32),
          pltpu.VMEM((MB, C, D), jnp.float32),
          pltpu.VMEM((MB, C, DE), jnp.float32),
          pltpu.VMEM((C, 8), jnp.float32),
          pltpu.SemaphoreType.DMA,
          pltpu.SemaphoreType.DMA,
          pltpu.SemaphoreType.DMA,
          pltpu.SemaphoreType.DMA,
          pltpu.SemaphoreType.DMA,
          pltpu.SemaphoreType.DMA,
          pltpu.SemaphoreType.DMA,
          pltpu.SemaphoreType.DMA,
          pltpu.SemaphoreType.DMA,
          pltpu.SemaphoreType.DMA,
          pltpu.SemaphoreType.DMA,
          pltpu.VMEM_SHARED((N, D), jnp.float32),
          pltpu.VMEM_SHARED((N, DE), jnp.float32),
          pltpu.VMEM_SHARED((N, 8), jnp.float32),
      ],
  )
  return fn(x, ei, ea, zd, ones)


def _sc_spmm(tbl, ei):
  def body(tbl_hbm, ei_hbm, s_out,
           src_v, dst_v, rows_v, rsem0, rsem1, rsem2,
           ssem0, ssem1, ssem2, dsem, s_sh):
    c = lax.axis_index("c")
    s = lax.axis_index("s")
    wid = c * NS + s
    rsems = (rsem0, rsem1, rsem2)
    ssems = (ssem0, ssem1, ssem2)

    _zero_rows(rows_v.at[0], C)

    @pl.when(c == 0)
    def _():
      pltpu.sync_copy(tbl_hbm.at[pl.ds(s * RPS, RPS)], s_sh.at[pl.ds(s * RPS, RPS)])

    @pl.when(c != 0)
    def _():
      _zero_stripe(s_sh, s * RPS, rows_v.at[0], D)

    pltpu.sync_copy(ei_hbm.at[0, pl.ds(wid * EPW, EPW)], src_v)
    plsc.subcore_barrier()

    def batch(i0, nb):
      dds = [pltpu.async_copy(ei_hbm.at[1, pl.ds((wid * CPW + i0 + b) * C, C)],
                              dst_v.at[b], dsem) for b in range(nb)]
      rds = [pltpu.async_copy(tbl_hbm.at[src_v.at[pl.ds((i0 + b) * C, C)]],
                              rows_v.at[b], rsems[b]) for b in range(nb)]
      for dd in dds:
        dd.wait()
      sds = []
      for b in range(nb):
        rds[b].wait()
        sds.append(pltpu.async_copy(rows_v.at[b], s_sh.at[dst_v.at[b]],
                                    ssems[b], add=True))
      for sd in sds:
        sd.wait()

    def outer(i, carry):
      batch(i * NBUF, NBUF)
      return carry

    lax.fori_loop(0, CPW // NBUF, outer, 0)
    if CPW % NBUF:
      batch(CPW - CPW % NBUF, CPW % NBUF)

    plsc.subcore_barrier()
    pltpu.sync_copy(s_sh.at[pl.ds(s * RPS, RPS)], s_out.at[c, pl.ds(s * RPS, RPS)])

  fn = pl.kernel(
      body,
      out_type=jax.ShapeDtypeStruct((NC, N, D), jnp.float32),
      mesh=_mesh,
      compiler_params=_sc_params,
      scratch_types=[
          pltpu.VMEM((EPW,), jnp.int32),
          pltpu.VMEM((NBUF, C), jnp.int32),
          pltpu.VMEM((NBUF, C, D), jnp.float32),
          pltpu.SemaphoreType.DMA,
          pltpu.SemaphoreType.DMA,
          pltpu.SemaphoreType.DMA,
          pltpu.SemaphoreType.DMA,
          pltpu.SemaphoreType.DMA,
          pltpu.SemaphoreType.DMA,
          pltpu.SemaphoreType.DMA,
          pltpu.VMEM_SHARED((N, D), jnp.float32),
      ],
  )
  return fn(tbl, ei)


NBK = 10
BLK = N // NBK


def _tcA_body(s_ref, t_ref, d_ref, st_ref, gin_ref, bin_ref, lwt_ref, ewt_ref,
              lbeb_ref, lb_ref, bias_ref, r_ref, sto_ref, acc_ref):
  i = pl.program_id(0)
  m = st_ref[0:1, :] * (1.0 / N)
  v = st_ref[1:2, :] * (1.0 / N) - m * m
  alpha = gin_ref[...] * lax.rsqrt(v + 1e-5)
  beta = bin_ref[...] - m * alpha
  deg = (d_ref[0] + d_ref[1])[:, 0:1]
  a = (s_ref[0] + s_ref[1]) * alpha + (deg + 1.0) * beta
  aggr = jnp.dot(a, lwt_ref[...], preferred_element_type=jnp.float32)
  aggr = aggr + jnp.dot(t_ref[0] + t_ref[1], ewt_ref[...],
                        preferred_element_type=jnp.float32)
  aggr = aggr + deg * lbeb_ref[...] + lb_ref[...]
  r = jnp.maximum(aggr, 0.0) + bias_ref[...]
  r_ref[...] = r

  @pl.when(i == 0)
  def _():
    acc_ref[...] = jnp.zeros((8, H), jnp.float32)

  acc_ref[0:1, :] += jnp.sum(r, axis=0, keepdims=True)
  acc_ref[1:2, :] += jnp.sum(r * r, axis=0, keepdims=True)

  @pl.when(i == NBK - 1)
  def _():
    sto_ref[...] = acc_ref[...]


def _tc_layer_raw(sp, tp, dp, st_in, g_in, b_in, lw, lb, ew, eb, bias):
  full = lambda shape: pl.BlockSpec(shape, lambda i: (0,) * len(shape))
  return pl.pallas_call(
      _tcA_body,
      grid=(NBK,),
      in_specs=[
          pl.BlockSpec((NC, BLK, D), lambda i: (0, i, 0)),
          pl.BlockSpec((NC, BLK, DE), lambda i: (0, i, 0)),
          pl.BlockSpec((NC, BLK, 8), lambda i: (0, i, 0)),
          full((8, H)),
          full((1, H)),
          full((1, H)),
          full((D, H)),
          full((DE, H)),
          full((1, H)),
          full((1, H)),
          full((1, H)),
      ],
      out_specs=[
          pl.BlockSpec((BLK, H), lambda i: (i, 0)),
          pl.BlockSpec((8, H), lambda i: (0, 0)),
      ],
      out_shape=[
          jax.ShapeDtypeStruct((N, H), jnp.float32),
          jax.ShapeDtypeStruct((8, H), jnp.float32),
      ],
      scratch_shapes=[pltpu.VMEM((8, H), jnp.float32)],
  )(sp, tp, dp, st_in, g_in.reshape(1, H), b_in.reshape(1, H), lw.T, ew.T,
    (lb + eb).reshape(1, H), lb.reshape(1, H), bias.reshape(1, H))


def _tcB_body(r_ref, st_ref, g_ref, b_ref, o_ref):
  m = st_ref[0:1, :] * (1.0 / N)
  v = st_ref[1:2, :] * (1.0 / N) - m * m
  alpha = g_ref[...] * lax.rsqrt(v + 1e-5)
  o_ref[...] = r_ref[...] * alpha + (b_ref[...] - m * alpha)


def _tc_norm(r, st, g, b):
  full = lambda shape: pl.BlockSpec(shape, lambda i: (0,) * len(shape))
  return pl.pallas_call(
      _tcB_body,
      grid=(NBK,),
      in_specs=[
          pl.BlockSpec((BLK, H), lambda i: (i, 0)),
          full((8, H)),
          full((1, H)),
          full((1, H)),
      ],
      out_specs=pl.BlockSpec((BLK, H), lambda i: (i, 0)),
      out_shape=jax.ShapeDtypeStruct((N, H), jnp.float32),
  )(r, st, g.reshape(1, H), b.reshape(1, H))


def kernel(x, edge_index, edge_attr, lin1_w, lin1_b, edge1_w, edge1_b, bias1,
           bn1_g, bn1_b, lin2_w, lin2_b, edge2_w, edge2_b, bias2, bn2_g, bn2_b):
  ei = edge_index.astype(jnp.int32)
  zd = jnp.zeros((N, 8), jnp.float32)
  ones = jnp.ones((C, 8), jnp.float32)
  st0 = jnp.zeros((8, H), jnp.float32).at[1].set(N * (1.0 - 1e-5))
  one_g = jnp.ones((H,), jnp.float32)
  zero_b = jnp.zeros((H,), jnp.float32)

  s1p, tp, dp = _sc_pass1(x, ei, edge_attr, zd, ones)
  r1, st1 = _tc_layer_raw(s1p, tp, dp, st0, one_g, zero_b,
                          lin1_w, lin1_b, edge1_w, edge1_b, bias1)
  s2p = _sc_spmm(r1, ei)
  r2, st2 = _tc_layer_raw(s2p, tp, dp, st1, bn1_g, bn1_b,
                          lin2_w, lin2_b, edge2_w, edge2_b, bias2)
  return _tc_norm(r2, st2, bn2_g, bn2_b)


# speedup vs baseline: 1.0128x; 1.0128x over previous
import jax
import jax.numpy as jnp
from jax import lax
from jax.experimental import pallas as pl
from jax.experimental.pallas import tpu as pltpu
from jax.experimental.pallas import tpu_sc as plsc

N = 10000
E = 320000
D = 128
DE = 16
H = 128

NC = 2
NS = 16
NW = NC * NS
C = 80
NCHUNKS = E // C
CPW = NCHUNKS // NW
RPS = N // NS
NBUF = 3
EBUF = 5

_mesh = plsc.VectorSubcoreMesh(core_axis_name="c", subcore_axis_name="s")
_sc_params = pltpu.CompilerParams(use_tc_tiling_on_sc=False)


MB = 2


EPW = E // NW


def _zero_rows(buf, nrows):
  zv = jnp.zeros((16,), jnp.float32)

  def zrow(r, carry):
    for k in range(8):
      buf[r, pl.ds(k * 16, 16)] = zv
    return carry

  lax.fori_loop(0, nrows, zrow, 0)


def _zero_stripe(sh, base, zsrc, width):
  for k in range(RPS // C):
    pltpu.sync_copy(zsrc, sh.at[pl.ds(base + k * C, C)])
  rem = RPS % C
  if rem:
    pltpu.sync_copy(zsrc.at[pl.ds(0, rem)],
                    sh.at[pl.ds(base + (RPS // C) * C, rem)])


def _sc_pass1(x, ei, ea, zd, ones):
  def body(x_hbm, ei_hbm, ea_hbm, zd_hbm, ones_hbm,
           s_out, t_out, d_out,
           src_v, dst_v, rows_v, ea_v, ones_v,
           rsem0, rsem1, ssem0, ssem1, esem0, esem1, tsem0, tsem1,
           osem0, osem1, dsem,
           s_sh, t_sh, d_sh):
    c = lax.axis_index("c")
    s = lax.axis_index("s")
    wid = c * NS + s
    rsems = (rsem0, rsem1)
    ssems = (ssem0, ssem1)
    esems = (esem0, esem1)
    tsems = (tsem0, tsem1)
    osems = (osem0, osem1)

    _zero_rows(rows_v.at[0], C)

    def zear(r, carry):
      ea_v[0, r, :] = jnp.zeros((16,), jnp.float32)
      return carry

    lax.fori_loop(0, C, zear, 0)

    @pl.when(c == 0)
    def _():
      pltpu.sync_copy(x_hbm.at[pl.ds(s * RPS, RPS)], s_sh.at[pl.ds(s * RPS, RPS)])

    @pl.when(c != 0)
    def _():
      _zero_stripe(s_sh, s * RPS, rows_v.at[0], D)

    _zero_stripe(t_sh, s * RPS, ea_v.at[0], DE)
    pltpu.sync_copy(zd_hbm.at[pl.ds(s * RPS, RPS)], d_sh.at[pl.ds(s * RPS, RPS)])
    pltpu.sync_copy(ei_hbm.at[0, pl.ds(wid * EPW, EPW)], src_v)
    pltpu.sync_copy(ones_hbm, ones_v)
    plsc.subcore_barrier()

    def batch(i0, nb):
      dds = [pltpu.async_copy(ei_hbm.at[1, pl.ds((wid * CPW + i0 + b) * C, C)],
                              dst_v.at[b], dsem) for b in range(nb)]
      rds = [pltpu.async_copy(x_hbm.at[src_v.at[pl.ds((i0 + b) * C, C)]],
                              rows_v.at[b], rsems[b]) for b in range(nb)]
      eds = [pltpu.async_copy(ea_hbm.at[pl.ds((wid * CPW + i0 + b) * C, C)],
                              ea_v.at[b], esems[b]) for b in range(nb)]
      for dd in dds:
        dd.wait()
      sds = []
      for b in range(nb):
        rds[b].wait()
        sds.append(pltpu.async_copy(rows_v.at[b], s_sh.at[dst_v.at[b]],
                                    ssems[b], add=True))
        eds[b].wait()
        sds.append(pltpu.async_copy(ea_v.at[b], t_sh.at[dst_v.at[b]],
                                    tsems[b], add=True))
        sds.append(pltpu.async_copy(ones_v, d_sh.at[dst_v.at[b]],
                                    osems[b], add=True))
      for sd in sds:
        sd.wait()

    def outer(i, carry):
      batch(i * MB, MB)
      return carry

    lax.fori_loop(0, CPW // MB, outer, 0)
    if CPW % MB:
      batch(CPW - CPW % MB, CPW % MB)

    plsc.subcore_barrier()
    pltpu.sync_copy(s_sh.at[pl.ds(s * RPS, RPS)], s_out.at[c, pl.ds(s * RPS, RPS)])
    pltpu.sync_copy(t_sh.at[pl.ds(s * RPS, RPS)], t_out.at[c, pl.ds(s * RPS, RPS)])
    pltpu.sync_copy(d_sh.at[pl.ds(s * RPS, RPS)], d_out.at[c, pl.ds(s * RPS, RPS)])

  fn = pl.kernel(
      body,
      out_type=[
          jax.ShapeDtypeStruct((NC, N, D), jnp.float32),
          jax.ShapeDtypeStruct((NC, N, DE), jnp.float32),
          jax.ShapeDtypeStruct((NC, N, 8), jnp.float32),
      ],
      mesh=_mesh,
      compiler_params=_sc_params,
      scratch_types=[
          pltpu.VMEM((EPW,), jnp.int32),
          pltpu.VMEM((MB, C), jnp.int32),
          pltpu.VMEM((MB, C, D), jnp.float32),
          pltpu.VMEM((MB, C, DE), jnp.float32),
          pltpu.VMEM((C, 8), jnp.float32),
          pltpu.SemaphoreType.DMA,
          pltpu.SemaphoreType.DMA,
          pltpu.SemaphoreType.DMA,
          pltpu.SemaphoreType.DMA,
          pltpu.SemaphoreType.DMA,
          pltpu.SemaphoreType.DMA,
          pltpu.SemaphoreType.DMA,
          pltpu.SemaphoreType.DMA,
          pltpu.SemaphoreType.DMA,
          pltpu.SemaphoreType.DMA,
          pltpu.SemaphoreType.DMA,
          pltpu.VMEM_SHARED((N, D), jnp.float32),
          pltpu.VMEM_SHARED((N, DE), jnp.float32),
          pltpu.VMEM_SHARED((N, 8), jnp.float32),
      ],
  )
  return fn(x, ei, ea, zd, ones)


def _sc_spmm(tbl, ei):
  def body(tbl_hbm, ei_hbm, s_out,
           src_v, dst_v, rows_v, rsem0, rsem1, rsem2,
           ssem0, ssem1, ssem2, dsem, s_sh):
    c = lax.axis_index("c")
    s = lax.axis_index("s")
    wid = c * NS + s
    rsems = (rsem0, rsem1, rsem2)
    ssems = (ssem0, ssem1, ssem2)

    _zero_rows(rows_v.at[0], C)

    @pl.when(c == 0)
    def _():
      pltpu.sync_copy(tbl_hbm.at[pl.ds(s * RPS, RPS)], s_sh.at[pl.ds(s * RPS, RPS)])

    @pl.when(c != 0)
    def _():
      _zero_stripe(s_sh, s * RPS, rows_v.at[0], D)

    pltpu.sync_copy(ei_hbm.at[0, pl.ds(wid * EPW, EPW)], src_v)
    plsc.subcore_barrier()

    def batch(i0, nb):
      dds = [pltpu.async_copy(ei_hbm.at[1, pl.ds((wid * CPW + i0 + b) * C, C)],
                              dst_v.at[b], dsem) for b in range(nb)]
      rds = [pltpu.async_copy(tbl_hbm.at[src_v.at[pl.ds((i0 + b) * C, C)]],
                              rows_v.at[b], rsems[b]) for b in range(nb)]
      for dd in dds:
        dd.wait()
      sds = []
      for b in range(nb):
        rds[b].wait()
        sds.append(pltpu.async_copy(rows_v.at[b], s_sh.at[dst_v.at[b]],
                                    ssems[b], add=True))
      for sd in sds:
        sd.wait()

    def outer(i, carry):
      batch(i * NBUF, NBUF)
      return carry

    lax.fori_loop(0, CPW // NBUF, outer, 0)
    if CPW % NBUF:
      batch(CPW - CPW % NBUF, CPW % NBUF)

    plsc.subcore_barrier()
    pltpu.sync_copy(s_sh.at[pl.ds(s * RPS, RPS)], s_out.at[c, pl.ds(s * RPS, RPS)])

  fn = pl.kernel(
      body,
      out_type=jax.ShapeDtypeStruct((NC, N, D), jnp.float32),
      mesh=_mesh,
      compiler_params=_sc_params,
      scratch_types=[
          pltpu.VMEM((EPW,), jnp.int32),
          pltpu.VMEM((NBUF, C), jnp.int32),
          pltpu.VMEM((NBUF, C, D), jnp.float32),
          pltpu.SemaphoreType.DMA,
          pltpu.SemaphoreType.DMA,
          pltpu.SemaphoreType.DMA,
          pltpu.SemaphoreType.DMA,
          pltpu.SemaphoreType.DMA,
          pltpu.SemaphoreType.DMA,
          pltpu.SemaphoreType.DMA,
          pltpu.VMEM_SHARED((N, D), jnp.float32),
      ],
  )
  return fn(tbl, ei)


NBK = 10
BLK = N // NBK


def _tcA_body(s_ref, t_ref, d_ref, st_ref, gin_ref, bin_ref, lwt_ref, ewt_ref,
              lbeb_ref, lb_ref, bias_ref, r_ref, sto_ref, acc_ref):
  i = pl.program_id(0)
  m = st_ref[0:1, :] * (1.0 / N)
  v = st_ref[1:2, :] * (1.0 / N) - m * m
  alpha = gin_ref[...] * lax.rsqrt(v + 1e-5)
  beta = bin_ref[...] - m * alpha
  deg = (d_ref[0] + d_ref[1])[:, 0:1]
  a = (s_ref[0] + s_ref[1]) * alpha + (deg + 1.0) * beta
  aggr = jnp.dot(a, lwt_ref[...], preferred_element_type=jnp.float32)
  aggr = aggr + jnp.dot(t_ref[0] + t_ref[1], ewt_ref[...],
                        preferred_element_type=jnp.float32)
  aggr = aggr + deg * lbeb_ref[...] + lb_ref[...]
  r = jnp.maximum(aggr, 0.0) + bias_ref[...]
  r_ref[...] = r

  @pl.when(i == 0)
  def _():
    acc_ref[...] = jnp.zeros((8, H), jnp.float32)

  acc_ref[0:1, :] += jnp.sum(r, axis=0, keepdims=True)
  acc_ref[1:2, :] += jnp.sum(r * r, axis=0, keepdims=True)

  @pl.when(i == NBK - 1)
  def _():
    sto_ref[...] = acc_ref[...]


def _tc_layer_raw(sp, tp, dp, st_in, g_in, b_in, lw, lb, ew, eb, bias):
  full = lambda shape: pl.BlockSpec(shape, lambda i: (0,) * len(shape))
  return pl.pallas_call(
      _tcA_body,
      grid=(NBK,),
      in_specs=[
          pl.BlockSpec((NC, BLK, D), lambda i: (0, i, 0)),
          pl.BlockSpec((NC, BLK, DE), lambda i: (0, i, 0)),
          pl.BlockSpec((NC, BLK, 8), lambda i: (0, i, 0)),
          full((8, H)),
          full((1, H)),
          full((1, H)),
          full((D, H)),
          full((DE, H)),
          full((1, H)),
          full((1, H)),
          full((1, H)),
      ],
      out_specs=[
          pl.BlockSpec((BLK, H), lambda i: (i, 0)),
          pl.BlockSpec((8, H), lambda i: (0, 0)),
      ],
      out_shape=[
          jax.ShapeDtypeStruct((N, H), jnp.float32),
          jax.ShapeDtypeStruct((8, H), jnp.float32),
      ],
      scratch_shapes=[pltpu.VMEM((8, H), jnp.float32)],
  )(sp, tp, dp, st_in, g_in.reshape(1, H), b_in.reshape(1, H), lw.T, ew.T,
    (lb + eb).reshape(1, H), lb.reshape(1, H), bias.reshape(1, H))


def _tcC_body(s_ref, t_ref, d_ref, st_ref, gin_ref, bin_ref, lwt_ref, ewt_ref,
              lbeb_ref, lb_ref, bias_ref, g_ref, b_ref, o_ref):
  m1 = st_ref[0:1, :] * (1.0 / N)
  v1 = st_ref[1:2, :] * (1.0 / N) - m1 * m1
  alpha = gin_ref[...] * lax.rsqrt(v1 + 1e-5)
  beta = bin_ref[...] - m1 * alpha
  deg = (d_ref[0] + d_ref[1])[:, 0:1]
  a = (s_ref[0] + s_ref[1]) * alpha + (deg + 1.0) * beta
  aggr = jnp.dot(a, lwt_ref[...], preferred_element_type=jnp.float32)
  aggr = aggr + jnp.dot(t_ref[0] + t_ref[1], ewt_ref[...],
                        preferred_element_type=jnp.float32)
  aggr = aggr + deg * lbeb_ref[...] + lb_ref[...]
  r = jnp.maximum(aggr, 0.0) + bias_ref[...]
  m = jnp.mean(r, axis=0, keepdims=True)
  cen = r - m
  v = jnp.mean(cen * cen, axis=0, keepdims=True)
  o_ref[...] = cen * lax.rsqrt(v + 1e-5) * g_ref[...] + b_ref[...]


def _tc_layer_final(sp, tp, dp, st_in, g_in, b_in, lw, lb, ew, eb, bias, g, b):
  return pl.pallas_call(
      _tcC_body,
      out_shape=jax.ShapeDtypeStruct((N, H), jnp.float32),
  )(sp, tp, dp, st_in, g_in.reshape(1, H), b_in.reshape(1, H), lw.T, ew.T,
    (lb + eb).reshape(1, H), lb.reshape(1, H), bias.reshape(1, H),
    g.reshape(1, H), b.reshape(1, H))


def kernel(x, edge_index, edge_attr, lin1_w, lin1_b, edge1_w, edge1_b, bias1,
           bn1_g, bn1_b, lin2_w, lin2_b, edge2_w, edge2_b, bias2, bn2_g, bn2_b):
  ei = edge_index.astype(jnp.int32)
  zd = jnp.zeros((N, 8), jnp.float32)
  ones = jnp.ones((C, 8), jnp.float32)
  st0 = jnp.zeros((8, H), jnp.float32).at[1].set(N * (1.0 - 1e-5))
  one_g = jnp.ones((H,), jnp.float32)
  zero_b = jnp.zeros((H,), jnp.float32)

  s1p, tp, dp = _sc_pass1(x, ei, edge_attr, zd, ones)
  r1, st1 = _tc_layer_raw(s1p, tp, dp, st0, one_g, zero_b,
                          lin1_w, lin1_b, edge1_w, edge1_b, bias1)
  s2p = _sc_spmm(r1, ei)
  return _tc_layer_final(s2p, tp, dp, st1, bn1_g, bn1_b,
                         lin2_w, lin2_b, edge2_w, edge2_b, bias2,
                         bn2_g, bn2_b)
